# Initial kernel scaffold; baseline (speedup 1.0000x reference)
import jax
import jax.numpy as jnp
from jax.experimental import pallas as pl


def kernel(hidden_states, weight):
    raise NotImplementedError("write your pallas kernel here")


# speedup vs baseline: 1.1114x; 1.1114x over previous
import jax
import jax.numpy as jnp
from jax.experimental import pallas as pl

_N_EXPERTS = 64
_TOP_K = 8


def _gate_kernel(x_ref, wt_ref, idx_ref, w_ref):
    x = x_ref[...]
    wt = wt_ref[...]
    scores = jnp.dot(x, wt, preferred_element_type=jnp.float32)
    m = jnp.max(scores, axis=-1, keepdims=True)
    e = jnp.exp(scores - m)
    probs = e / jnp.sum(e, axis=-1, keepdims=True)

    iota = jax.lax.broadcasted_iota(jnp.int32, probs.shape, 1)
    s = probs
    vals, idxs = [], []
    for _ in range(_TOP_K):
        mk = jnp.max(s, axis=-1, keepdims=True)
        ik = jnp.min(jnp.where(s == mk, iota, _N_EXPERTS),
                     axis=-1, keepdims=True)
        vals.append(mk)
        idxs.append(ik)
        s = jnp.where(iota == ik, -1.0, s)
    w = jnp.concatenate(vals, axis=-1)
    ids = jnp.concatenate(idxs, axis=-1)
    w = w / jnp.sum(w, axis=-1, keepdims=True)
    idx_ref[...] = ids
    w_ref[...] = w


def kernel(hidden_states, weight):
    x = hidden_states.reshape(-1, hidden_states.shape[-1])
    n, h = x.shape
    wt = weight.T
    t = 1024
    idx, w = pl.pallas_call(
        _gate_kernel,
        grid=(n // t,),
        in_specs=[
            pl.BlockSpec((t, h), lambda i: (i, 0)),
            pl.BlockSpec((h, _N_EXPERTS), lambda i: (0, 0)),
        ],
        out_specs=[
            pl.BlockSpec((t, _TOP_K), lambda i: (i, 0)),
            pl.BlockSpec((t, _TOP_K), lambda i: (i, 0)),
        ],
        out_shape=[
            jax.ShapeDtypeStruct((n, _TOP_K), jnp.int32),
            jax.ShapeDtypeStruct((n, _TOP_K), jnp.float32),
        ],
    )(x, wt)
    return idx, w


# speedup vs baseline: 2.1598x; 1.9433x over previous
import jax
import jax.numpy as jnp
from jax.experimental import pallas as pl

_N_EXPERTS = 64
_TOP_K = 8


def _gate_kernel(x_ref, wt_ref, idx_ref, w_ref):
    x = x_ref[...]
    wt = wt_ref[...]
    scores = jnp.dot(x, wt, preferred_element_type=jnp.float32)
    st = scores.T

    t = st.shape[1]
    iota = jax.lax.broadcasted_iota(jnp.int32, (_N_EXPERTS, t), 0)
    s = st
    vals, idxs = [], []
    for k in range(_TOP_K):
        mk = jnp.max(s, axis=0, keepdims=True)
        ik = (_N_EXPERTS - 1) - jnp.max(
            jnp.where(s == mk, (_N_EXPERTS - 1) - iota, -1),
            axis=0, keepdims=True)
        vals.append(mk)
        idxs.append(ik)
        if k < _TOP_K - 1:
            s = jnp.where(iota == ik, -1e30, s)

    v = jnp.concatenate(vals, axis=0)
    e = jnp.exp(v - vals[0])
    w = e / jnp.sum(e, axis=0, keepdims=True)
    ids = jnp.concatenate(idxs, axis=0)
    idx_ref[...] = ids.T
    w_ref[...] = w.T


def kernel(hidden_states, weight):
    x = hidden_states.reshape(-1, hidden_states.shape[-1])
    n, h = x.shape
    wt = weight.T
    t = 1024
    idx, w = pl.pallas_call(
        _gate_kernel,
        grid=(n // t,),
        in_specs=[
            pl.BlockSpec((t, h), lambda i: (i, 0)),
            pl.BlockSpec((h, _N_EXPERTS), lambda i: (0, 0)),
        ],
        out_specs=[
            pl.BlockSpec((t, _TOP_K), lambda i: (i, 0)),
            pl.BlockSpec((t, _TOP_K), lambda i: (i, 0)),
        ],
        out_shape=[
            jax.ShapeDtypeStruct((n, _TOP_K), jnp.int32),
            jax.ShapeDtypeStruct((n, _TOP_K), jnp.float32),
        ],
    )(x, wt)
    return idx, w


# speedup vs baseline: 3.9023x; 1.8068x over previous
import jax
import jax.numpy as jnp
from jax.experimental import pallas as pl

_N_EXPERTS = 64
_TOP_K = 8


def _gate_kernel(x_ref, wt_ref, idx_ref, w_ref):
    x = x_ref[...]
    wt = wt_ref[...]
    scores = jnp.dot(x, wt, preferred_element_type=jnp.float32)
    st = scores.T

    t = st.shape[1]
    iota = jax.lax.broadcasted_iota(jnp.int32, (_N_EXPERTS, t), 0)
    s = st
    vals, idxs = [], []
    for k in range(_TOP_K):
        mk = jnp.max(s, axis=0, keepdims=True)
        ik = (_N_EXPERTS - 1) - jnp.max(
            jnp.where(s == mk, (_N_EXPERTS - 1) - iota, -1),
            axis=0, keepdims=True)
        vals.append(mk)
        idxs.append(ik)
        if k < _TOP_K - 1:
            s = jnp.where(iota == ik, -1e30, s)

    v = jnp.concatenate(vals, axis=0)
    e = jnp.exp(v - vals[0])
    w = e / jnp.sum(e, axis=0, keepdims=True)
    ids = jnp.concatenate(idxs, axis=0)
    idx_ref[...] = ids
    w_ref[...] = w


def kernel(hidden_states, weight):
    x = hidden_states.reshape(-1, hidden_states.shape[-1])
    n, h = x.shape
    wt = weight.T
    t = 2048
    idx_t, w_t = pl.pallas_call(
        _gate_kernel,
        grid=(n // t,),
        in_specs=[
            pl.BlockSpec((t, h), lambda i: (i, 0)),
            pl.BlockSpec((h, _N_EXPERTS), lambda i: (0, 0)),
        ],
        out_specs=[
            pl.BlockSpec((_TOP_K, t), lambda i: (0, i)),
            pl.BlockSpec((_TOP_K, t), lambda i: (0, i)),
        ],
        out_shape=[
            jax.ShapeDtypeStruct((_TOP_K, n), jnp.int32),
            jax.ShapeDtypeStruct((_TOP_K, n), jnp.float32),
        ],
    )(x, wt)
    return idx_t.T, w_t.T


# speedup vs baseline: 4.3322x; 1.1102x over previous
import jax
import jax.numpy as jnp
from jax.experimental import pallas as pl

_N_EXPERTS = 64
_TOP_K = 8


def _gate_kernel(x_ref, wt_ref, idx_ref, w_ref):
    x = x_ref[...]
    wt = wt_ref[...]
    scores = jnp.dot(x, wt, preferred_element_type=jnp.float32)
    st = scores.T
    t = st.shape[1]

    b = jax.lax.bitcast_convert_type(st, jnp.int32)
    mono = b ^ jax.lax.shift_right_logical(
        jax.lax.shift_right_arithmetic(b, 31), 1)
    rev_iota = (_N_EXPERTS - 1) - jax.lax.broadcasted_iota(
        jnp.int32, (_N_EXPERTS, t), 0)
    key = (mono & jnp.int32(~(_N_EXPERTS - 1))) | rev_iota

    picks = []
    for k in range(_TOP_K):
        mk = jnp.max(key, axis=0, keepdims=True)
        picks.append(mk)
        if k < _TOP_K - 1:
            key = jnp.where(key == mk, jnp.int32(-2147483648), key)

    pk = jnp.concatenate(picks, axis=0)
    ids = (_N_EXPERTS - 1) - (pk & jnp.int32(_N_EXPERTS - 1))
    vb = pk & jnp.int32(~(_N_EXPERTS - 1))
    vb = vb ^ jax.lax.shift_right_logical(
        jax.lax.shift_right_arithmetic(vb, 31), 1)
    v = jax.lax.bitcast_convert_type(vb, jnp.float32)
    e = jnp.exp(v - v[0:1, :])
    w = e / jnp.sum(e, axis=0, keepdims=True)
    idx_ref[...] = ids
    w_ref[...] = w


def kernel(hidden_states, weight):
    x = hidden_states.reshape(-1, hidden_states.shape[-1])
    n, h = x.shape
    wt = weight.T
    t = 2048
    idx_t, w_t = pl.pallas_call(
        _gate_kernel,
        grid=(n // t,),
        in_specs=[
            pl.BlockSpec((t, h), lambda i: (i, 0)),
            pl.BlockSpec((h, _N_EXPERTS), lambda i: (0, 0)),
        ],
        out_specs=[
            pl.BlockSpec((_TOP_K, t), lambda i: (0, i)),
            pl.BlockSpec((_TOP_K, t), lambda i: (0, i)),
        ],
        out_shape=[
            jax.ShapeDtypeStruct((_TOP_K, n), jnp.int32),
            jax.ShapeDtypeStruct((_TOP_K, n), jnp.float32),
        ],
    )(x, wt)
    return idx_t.T, w_t.T


# speedup vs baseline: 4.7129x; 1.0879x over previous
import jax
import jax.numpy as jnp
from jax.experimental import pallas as pl

_N_EXPERTS = 64
_TOP_K = 8


def _gate_kernel(x_ref, wt_ref, idx_ref, w_ref):
    x = x_ref[...]
    wt = wt_ref[...]
    scores = jnp.dot(x, wt, preferred_element_type=jnp.float32)
    st = scores.T
    t = st.shape[1]

    b = jax.lax.bitcast_convert_type(st, jnp.int32)
    mono = b ^ jax.lax.shift_right_logical(
        jax.lax.shift_right_arithmetic(b, 31), 1)
    rev_iota = (_N_EXPERTS - 1) - jax.lax.broadcasted_iota(
        jnp.int32, (_N_EXPERTS, t), 0)
    key = (mono & jnp.int32(~(_N_EXPERTS - 1))) | rev_iota

    picks = []
    for k in range(_TOP_K):
        mk = jnp.max(key, axis=0, keepdims=True)
        picks.append(mk)
        if k < _TOP_K - 1:
            key = jnp.where(key == mk, jnp.int32(-2147483648), key)

    pk = jnp.concatenate(picks, axis=0)
    ids = (_N_EXPERTS - 1) - (pk & jnp.int32(_N_EXPERTS - 1))
    vb = pk & jnp.int32(~(_N_EXPERTS - 1))
    vb = vb ^ jax.lax.shift_right_logical(
        jax.lax.shift_right_arithmetic(vb, 31), 1)
    v = jax.lax.bitcast_convert_type(vb, jnp.float32)
    e = jnp.exp(v - v[0:1, :])
    w = e / jnp.sum(e, axis=0, keepdims=True)
    idx_ref[...] = ids
    w_ref[...] = w


def kernel(hidden_states, weight):
    x = hidden_states.reshape(-1, hidden_states.shape[-1])
    n, h = x.shape
    wt = weight.T
    t = 4096
    idx_t, w_t = pl.pallas_call(
        _gate_kernel,
        grid=(n // t,),
        in_specs=[
            pl.BlockSpec((t, h), lambda i: (i, 0)),
            pl.BlockSpec((h, _N_EXPERTS), lambda i: (0, 0)),
        ],
        out_specs=[
            pl.BlockSpec((_TOP_K, t), lambda i: (0, i)),
            pl.BlockSpec((_TOP_K, t), lambda i: (0, i)),
        ],
        out_shape=[
            jax.ShapeDtypeStruct((_TOP_K, n), jnp.int32),
            jax.ShapeDtypeStruct((_TOP_K, n), jnp.float32),
        ],
    )(x, wt)
    return idx_t.T, w_t.T
